# baseline (speedup 1.0000x reference)
import functools

import jax
import jax.numpy as jnp
from jax import lax
from jax.experimental import pallas as pl
from jax.experimental.pallas import tpu as pltpu
from jax.experimental.pallas import tpu_sc as plsc

N = 10000
D = 128
E = 320000
NC = 2
NS = 16
NW = NC * NS
K = 64
NCHUNK = 160
EPAD = NW * NCHUNK * K
ROWS = 10240
RPT = ROWS // NS
DUMMY = N
RB = 2000
GRID = N // RB

_mesh = plsc.VectorSubcoreMesh(core_axis_name="c", subcore_axis_name="s")


@functools.partial(
    pl.kernel,
    out_type=jax.ShapeDtypeStruct((NC, ROWS), jnp.float32),
    mesh=_mesh,
    scratch_types=[
        pltpu.VMEM_SHARED((ROWS,), jnp.float32),
        pltpu.VMEM((NCHUNK, K), jnp.int32),
        pltpu.VMEM((K,), jnp.float32),
        pltpu.VMEM((RPT,), jnp.float32),
    ],
)
def _deg_kernel(rowd_hbm, degp_hbm, deg_sh, idx_v, ones_v, zb_v):
    c = lax.axis_index("c")
    s = lax.axis_index("s")
    wid = c * NS + s
    ones16 = jnp.ones((16,), jnp.float32)
    zeros16 = jnp.zeros((16,), jnp.float32)
    for q in range(K // 16):
        ones_v[pl.ds(q * 16, 16)] = ones16
    for q in range(RPT // 16):
        zb_v[pl.ds(q * 16, 16)] = zeros16
    pltpu.sync_copy(zb_v, deg_sh.at[pl.ds(s * RPT, RPT)])
    pltpu.sync_copy(rowd_hbm.at[wid], idx_v)
    plsc.subcore_barrier()

    def body(j, carry):
        pltpu.sync_copy(ones_v, deg_sh.at[idx_v.at[j]], add=True)
        return carry

    lax.fori_loop(0, NCHUNK, body, 0)
    plsc.subcore_barrier()
    pltpu.sync_copy(deg_sh.at[pl.ds(s * RPT, RPT)],
                    degp_hbm.at[c, pl.ds(s * RPT, RPT)])


@functools.partial(
    pl.kernel,
    out_type=jax.ShapeDtypeStruct((NC, ROWS, D), jnp.float32),
    mesh=_mesh,
    scratch_types=[
        pltpu.VMEM_SHARED((ROWS, D), jnp.float32),
        pltpu.VMEM((2, K, D), jnp.float32),
        pltpu.VMEM((NCHUNK // 2, K), jnp.int32),
        pltpu.VMEM((NCHUNK // 2, K), jnp.int32),
        pltpu.VMEM((16, D), jnp.float32),
        pltpu.SemaphoreType.DMA,
        pltpu.SemaphoreType.DMA,
    ],
)
def _scatter_kernel(h_hbm, rowg_hbm, cols_hbm, sp_hbm,
                    acc_sh, gbuf, idxg, idxs, zb, semA, semB):
    c = lax.axis_index("c")
    s = lax.axis_index("s")
    wid = c * NS + s
    zeros16 = jnp.zeros((16,), jnp.float32)

    def zrow(i, carry):
        for q in range(D // 16):
            zb[i, pl.ds(q * 16, 16)] = zeros16
        return carry

    lax.fori_loop(0, 16, zrow, 0)
    for t in range(RPT // 16):
        pltpu.sync_copy(zb, acc_sh.at[pl.ds(s * RPT + t * 16, 16)])
    plsc.subcore_barrier()

    HC = NCHUNK // 2
    NG = HC // 2
    for half in range(2):
        pltpu.sync_copy(rowg_hbm.at[wid, pl.ds(half * HC, HC)], idxg)
        pltpu.sync_copy(cols_hbm.at[wid, pl.ds(half * HC, HC)], idxs)
        pltpu.async_copy(h_hbm.at[idxg.at[0]], gbuf.at[0], semA)
        pltpu.async_copy(h_hbm.at[idxg.at[1]], gbuf.at[1], semB)

        def body(g, carry):
            j0 = 2 * g
            pltpu.make_async_copy(
                h_hbm.at[idxg.at[0]], gbuf.at[0], semA).wait()
            pltpu.sync_copy(gbuf.at[0], acc_sh.at[idxs.at[j0]], add=True)

            @pl.when(g < NG - 1)
            def _():
                pltpu.async_copy(h_hbm.at[idxg.at[j0 + 2]], gbuf.at[0], semA)

            pltpu.make_async_copy(
                h_hbm.at[idxg.at[1]], gbuf.at[1], semB).wait()
            pltpu.sync_copy(gbuf.at[1], acc_sh.at[idxs.at[j0 + 1]], add=True)

            @pl.when(g < NG - 1)
            def _():
                pltpu.async_copy(h_hbm.at[idxg.at[j0 + 3]], gbuf.at[1], semB)

            return carry

        lax.fori_loop(0, NG, body, 0)
    plsc.subcore_barrier()
    pltpu.sync_copy(acc_sh.at[pl.ds(s * RPT, RPT)],
                    sp_hbm.at[c, pl.ds(s * RPT, RPT)])


def _tc1_body(x_ref, w1_ref, degt_ref, h1p_ref, dinv_ref):
    deg = degt_ref[:, 0:1] + degt_ref[:, 1:2] + 0.5
    dinv = lax.rsqrt(deg)
    h = jnp.dot(x_ref[...], w1_ref[...], preferred_element_type=jnp.float32)
    h1p_ref[...] = h * dinv
    dinv_ref[...] = dinv


def _tc1(x, W1, degt):
    return pl.pallas_call(
        _tc1_body,
        grid=(GRID,),
        in_specs=[
            pl.BlockSpec((RB, D), lambda i: (i, 0)),
            pl.BlockSpec((D, D), lambda i: (0, 0)),
            pl.BlockSpec((RB, NC), lambda i: (i, 0)),
        ],
        out_specs=[
            pl.BlockSpec((RB, D), lambda i: (i, 0)),
            pl.BlockSpec((RB, 1), lambda i: (i, 0)),
        ],
        out_shape=[
            jax.ShapeDtypeStruct((N, D), jnp.float32),
            jax.ShapeDtypeStruct((N, 1), jnp.float32),
        ],
    )(x, W1, degt)


def _tc2_body(sp_ref, h1p_ref, dinv_ref, b1_ref, w2_ref, h2p_ref):
    ssum = sp_ref[0] + sp_ref[1]
    dinv = dinv_ref[...]
    z = jnp.maximum(dinv * (ssum + 0.5 * h1p_ref[...]) + b1_ref[...], 0.0)
    h2 = jnp.dot(z, w2_ref[...], preferred_element_type=jnp.float32)
    h2p_ref[...] = h2 * dinv


def _tc2(sp1, h1p, dinv, b1, W2):
    return pl.pallas_call(
        _tc2_body,
        grid=(GRID,),
        in_specs=[
            pl.BlockSpec((NC, RB, D), lambda i: (0, i, 0)),
            pl.BlockSpec((RB, D), lambda i: (i, 0)),
            pl.BlockSpec((RB, 1), lambda i: (i, 0)),
            pl.BlockSpec((1, D), lambda i: (0, 0)),
            pl.BlockSpec((D, D), lambda i: (0, 0)),
        ],
        out_specs=pl.BlockSpec((RB, D), lambda i: (i, 0)),
        out_shape=jax.ShapeDtypeStruct((N, D), jnp.float32),
    )(sp1, h1p, dinv, b1, W2)


def _tc3_body(sp_ref, h2p_ref, dinv_ref, b2_ref, out_ref):
    ssum = sp_ref[0] + sp_ref[1]
    out_ref[...] = dinv_ref[...] * (ssum + 0.5 * h2p_ref[...]) + b2_ref[...]


def _tc3(sp2, h2p, dinv, b2):
    return pl.pallas_call(
        _tc3_body,
        grid=(GRID,),
        in_specs=[
            pl.BlockSpec((NC, RB, D), lambda i: (0, i, 0)),
            pl.BlockSpec((RB, D), lambda i: (i, 0)),
            pl.BlockSpec((RB, 1), lambda i: (i, 0)),
            pl.BlockSpec((1, D), lambda i: (0, 0)),
        ],
        out_specs=pl.BlockSpec((RB, D), lambda i: (i, 0)),
        out_shape=jax.ShapeDtypeStruct((N, D), jnp.float32),
    )(sp2, h2p, dinv, b2)


@jax.jit
def kernel(x, edge_index, W1, b1, W2, b2):
    row = edge_index[0]
    col = edge_index[1]
    pad = EPAD - E
    dummy = DUMMY + (jnp.arange(pad, dtype=jnp.int32) % (ROWS - N))
    rowg = jnp.concatenate(
        [row, jnp.zeros((pad,), jnp.int32)]).reshape(NW, NCHUNK, K)
    cols = jnp.concatenate([col, dummy]).reshape(NW, NCHUNK, K)
    rowd = jnp.concatenate([row, dummy]).reshape(NW, NCHUNK, K)

    degp = _deg_kernel(rowd)
    degt = degp.T
    h1p, dinv = _tc1(x, W1, degt)
    sp1 = _scatter_kernel(h1p, rowg, cols)
    h2p = _tc2(sp1, h1p, dinv, b1.reshape(1, D), W2)
    sp2 = _scatter_kernel(h2p, rowg, cols)
    return _tc3(sp2, h2p, dinv, b2.reshape(1, D))


# speedup vs baseline: 1.0314x; 1.0314x over previous
import functools

import jax
import jax.numpy as jnp
from jax import lax
from jax.experimental import pallas as pl
from jax.experimental.pallas import tpu as pltpu
from jax.experimental.pallas import tpu_sc as plsc

N = 10000
D = 128
E = 320000
NC = 2
NS = 16
K = 128
CPS = 160
EPAD = NS * CPS * K
NCHUNK0 = 40
IDXB = 32
ROWS = 10240
RPT = ROWS // NS
DUMMY = N
RB = 2000
GRID = N // RB

_mesh = plsc.VectorSubcoreMesh(core_axis_name="c", subcore_axis_name="s")


@functools.partial(
    pl.kernel,
    out_type=jax.ShapeDtypeStruct((NC, ROWS), jnp.float32),
    mesh=_mesh,
    scratch_types=[
        pltpu.VMEM_SHARED((ROWS,), jnp.float32),
        pltpu.VMEM((CPS // 2, K), jnp.int32),
        pltpu.VMEM((K,), jnp.float32),
        pltpu.VMEM((RPT,), jnp.float32),
    ],
)
def _deg_kernel(rowd_hbm, degp_hbm, deg_sh, idx_v, ones_v, zb_v):
    c = lax.axis_index("c")
    s = lax.axis_index("s")
    ones16 = jnp.ones((16,), jnp.float32)
    zeros16 = jnp.zeros((16,), jnp.float32)
    for q in range(K // 16):
        ones_v[pl.ds(q * 16, 16)] = ones16
    for q in range(RPT // 16):
        zb_v[pl.ds(q * 16, 16)] = zeros16
    pltpu.sync_copy(zb_v, deg_sh.at[pl.ds(s * RPT, RPT)])
    pltpu.sync_copy(rowd_hbm.at[s, pl.ds(c * (CPS // 2), CPS // 2)], idx_v)
    plsc.subcore_barrier()

    def body(j, carry):
        pltpu.sync_copy(ones_v, deg_sh.at[idx_v.at[j]], add=True)
        return carry

    lax.fori_loop(0, CPS // 2, body, 0)
    plsc.subcore_barrier()
    pltpu.sync_copy(deg_sh.at[pl.ds(s * RPT, RPT)],
                    degp_hbm.at[c, pl.ds(s * RPT, RPT)])


def _blocks(start, n):
    out = []
    while n > 0:
        b = min(n, IDXB)
        out.append((start, b))
        start += b
        n -= b
    return out


@functools.partial(
    pl.kernel,
    out_type=jax.ShapeDtypeStruct((NC, ROWS, D), jnp.float32),
    mesh=_mesh,
    scratch_types=[
        pltpu.VMEM_SHARED((ROWS, D), jnp.float32),
        pltpu.VMEM((2, K, D), jnp.float32),
        pltpu.VMEM((IDXB, K), jnp.int32),
        pltpu.VMEM((IDXB, K), jnp.int32),
        pltpu.VMEM((32, D), jnp.float32),
        pltpu.SemaphoreType.DMA,
        pltpu.SemaphoreType.DMA,
    ],
)
def _scatter_kernel(h_hbm, rowg_hbm, cols_hbm, sp_hbm,
                    acc_sh, gbuf, idxg, idxs, zb, semA, semB):
    c = lax.axis_index("c")
    s = lax.axis_index("s")
    zeros16 = jnp.zeros((16,), jnp.float32)

    def zrow(i, carry):
        for q in range(D // 16):
            zb[i, pl.ds(q * 16, 16)] = zeros16
        return carry

    lax.fori_loop(0, 32, zrow, 0)
    for t in range(RPT // 32):
        pltpu.sync_copy(zb, acc_sh.at[pl.ds(s * RPT + t * 32, 32)])
    plsc.subcore_barrier()

    def run_block(st, n):
        pltpu.sync_copy(rowg_hbm.at[s, pl.ds(st, n)], idxg.at[pl.ds(0, n)])
        pltpu.sync_copy(cols_hbm.at[s, pl.ds(st, n)], idxs.at[pl.ds(0, n)])
        pltpu.async_copy(h_hbm.at[idxg.at[0]], gbuf.at[0], semA)
        pltpu.async_copy(h_hbm.at[idxg.at[1]], gbuf.at[1], semB)
        ng = n // 2

        def body(g, carry):
            j0 = 2 * g
            pltpu.make_async_copy(
                h_hbm.at[idxg.at[0]], gbuf.at[0], semA).wait()
            pltpu.sync_copy(gbuf.at[0], acc_sh.at[idxs.at[j0]], add=True)

            @pl.when(g < ng - 1)
            def _():
                pltpu.async_copy(h_hbm.at[idxg.at[j0 + 2]], gbuf.at[0], semA)

            pltpu.make_async_copy(
                h_hbm.at[idxg.at[1]], gbuf.at[1], semB).wait()
            pltpu.sync_copy(gbuf.at[1], acc_sh.at[idxs.at[j0 + 1]], add=True)

            @pl.when(g < ng - 1)
            def _():
                pltpu.async_copy(h_hbm.at[idxg.at[j0 + 3]], gbuf.at[1], semB)

            return carry

        lax.fori_loop(0, ng, body, 0)

    @pl.when(c == 0)
    def _():
        for st, n in _blocks(0, NCHUNK0):
            run_block(st, n)

    @pl.when(c == 1)
    def _():
        for st, n in _blocks(NCHUNK0, CPS - NCHUNK0):
            run_block(st, n)

    plsc.subcore_barrier()
    pltpu.sync_copy(acc_sh.at[pl.ds(s * RPT, RPT)],
                    sp_hbm.at[c, pl.ds(s * RPT, RPT)])


def _tc1_body(x_ref, w1_ref, degt_ref, h1p_ref, dinv_ref):
    deg = degt_ref[:, 0:1] + degt_ref[:, 1:2] + 0.5
    dinv = lax.rsqrt(deg)
    h = jnp.dot(x_ref[...], w1_ref[...], preferred_element_type=jnp.float32)
    h1p_ref[...] = h * dinv
    dinv_ref[...] = dinv


def _tc1(x, W1, degt):
    return pl.pallas_call(
        _tc1_body,
        grid=(GRID,),
        in_specs=[
            pl.BlockSpec((RB, D), lambda i: (i, 0)),
            pl.BlockSpec((D, D), lambda i: (0, 0)),
            pl.BlockSpec((RB, NC), lambda i: (i, 0)),
        ],
        out_specs=[
            pl.BlockSpec((RB, D), lambda i: (i, 0)),
            pl.BlockSpec((RB, 1), lambda i: (i, 0)),
        ],
        out_shape=[
            jax.ShapeDtypeStruct((N, D), jnp.float32),
            jax.ShapeDtypeStruct((N, 1), jnp.float32),
        ],
    )(x, W1, degt)


def _tc2_body(sp_ref, h1p_ref, dinv_ref, b1_ref, w2_ref, h2p_ref):
    ssum = sp_ref[0] + sp_ref[1]
    dinv = dinv_ref[...]
    z = jnp.maximum(dinv * (ssum + 0.5 * h1p_ref[...]) + b1_ref[...], 0.0)
    h2 = jnp.dot(z, w2_ref[...], preferred_element_type=jnp.float32)
    h2p_ref[...] = h2 * dinv


def _tc2(sp1, h1p, dinv, b1, W2):
    return pl.pallas_call(
        _tc2_body,
        grid=(GRID,),
        in_specs=[
            pl.BlockSpec((NC, RB, D), lambda i: (0, i, 0)),
            pl.BlockSpec((RB, D), lambda i: (i, 0)),
            pl.BlockSpec((RB, 1), lambda i: (i, 0)),
            pl.BlockSpec((1, D), lambda i: (0, 0)),
            pl.BlockSpec((D, D), lambda i: (0, 0)),
        ],
        out_specs=pl.BlockSpec((RB, D), lambda i: (i, 0)),
        out_shape=jax.ShapeDtypeStruct((N, D), jnp.float32),
    )(sp1, h1p, dinv, b1, W2)


def _tc3_body(sp_ref, h2p_ref, dinv_ref, b2_ref, out_ref):
    ssum = sp_ref[0] + sp_ref[1]
    out_ref[...] = dinv_ref[...] * (ssum + 0.5 * h2p_ref[...]) + b2_ref[...]


def _tc3(sp2, h2p, dinv, b2):
    return pl.pallas_call(
        _tc3_body,
        grid=(GRID,),
        in_specs=[
            pl.BlockSpec((NC, RB, D), lambda i: (0, i, 0)),
            pl.BlockSpec((RB, D), lambda i: (i, 0)),
            pl.BlockSpec((RB, 1), lambda i: (i, 0)),
            pl.BlockSpec((1, D), lambda i: (0, 0)),
        ],
        out_specs=pl.BlockSpec((RB, D), lambda i: (i, 0)),
        out_shape=jax.ShapeDtypeStruct((N, D), jnp.float32),
    )(sp2, h2p, dinv, b2)


@jax.jit
def kernel(x, edge_index, W1, b1, W2, b2):
    row = edge_index[0]
    col = edge_index[1]
    pad = EPAD - E
    dummy = DUMMY + (jnp.arange(pad, dtype=jnp.int32) % (ROWS - N))
    rowg = jnp.concatenate(
        [row, jnp.zeros((pad,), jnp.int32)]).reshape(NS, CPS, K)
    cols = jnp.concatenate([col, dummy]).reshape(NS, CPS, K)
    rowd = jnp.concatenate([row, dummy]).reshape(NS, CPS, K)

    degp = _deg_kernel(rowd)
    degt = degp.T
    h1p, dinv = _tc1(x, W1, degt)
    sp1 = _scatter_kernel(h1p, rowg, cols)
    h2p = _tc2(sp1, h1p, dinv, b1.reshape(1, D), W2)
    sp2 = _scatter_kernel(h2p, rowg, cols)
    return _tc3(sp2, h2p, dinv, b2.reshape(1, D))


# speedup vs baseline: 1.0735x; 1.0408x over previous
import functools

import jax
import jax.numpy as jnp
from jax import lax
from jax.experimental import pallas as pl
from jax.experimental.pallas import tpu as pltpu
from jax.experimental.pallas import tpu_sc as plsc

N = 10000
D = 128
E = 320000
NC = 2
NS = 16
K = 128
CPS = 160
EPAD = NS * CPS * K
NCHUNK0 = 80
IDXB = 32
ROWS = 10240
RPT = ROWS // NS
DUMMY = N
RB = 2000
GRID = N // RB

_mesh = plsc.VectorSubcoreMesh(core_axis_name="c", subcore_axis_name="s")


@functools.partial(
    pl.kernel,
    out_type=jax.ShapeDtypeStruct((NC, ROWS), jnp.float32),
    mesh=_mesh,
    scratch_types=[
        pltpu.VMEM_SHARED((ROWS,), jnp.float32),
        pltpu.VMEM((CPS // 2, K), jnp.int32),
        pltpu.VMEM((K,), jnp.float32),
        pltpu.VMEM((RPT,), jnp.float32),
    ],
)
def _deg_kernel(rowd_hbm, degp_hbm, deg_sh, idx_v, ones_v, zb_v):
    c = lax.axis_index("c")
    s = lax.axis_index("s")
    ones16 = jnp.ones((16,), jnp.float32)
    zeros16 = jnp.zeros((16,), jnp.float32)
    for q in range(K // 16):
        ones_v[pl.ds(q * 16, 16)] = ones16
    for q in range(RPT // 16):
        zb_v[pl.ds(q * 16, 16)] = zeros16
    pltpu.sync_copy(zb_v, deg_sh.at[pl.ds(s * RPT, RPT)])
    pltpu.sync_copy(rowd_hbm.at[s, pl.ds(c * (CPS // 2), CPS // 2)], idx_v)
    plsc.subcore_barrier()

    def body(j, carry):
        pltpu.sync_copy(ones_v, deg_sh.at[idx_v.at[j]], add=True)
        return carry

    lax.fori_loop(0, CPS // 2, body, 0)
    plsc.subcore_barrier()
    pltpu.sync_copy(deg_sh.at[pl.ds(s * RPT, RPT)],
                    degp_hbm.at[c, pl.ds(s * RPT, RPT)])


def _blocks(start, n):
    out = []
    while n > 0:
        b = min(n, IDXB)
        out.append((start, b))
        start += b
        n -= b
    return out


@functools.partial(
    pl.kernel,
    out_type=jax.ShapeDtypeStruct((NC, ROWS, D), jnp.float32),
    mesh=_mesh,
    scratch_types=[
        pltpu.VMEM_SHARED((ROWS, D), jnp.float32),
        pltpu.VMEM((2, K, D), jnp.float32),
        pltpu.VMEM((IDXB, K), jnp.int32),
        pltpu.VMEM((IDXB, K), jnp.int32),
        pltpu.VMEM((32, D), jnp.float32),
        pltpu.SemaphoreType.DMA,
        pltpu.SemaphoreType.DMA,
    ],
)
def _scatter_kernel(h_hbm, rowg_hbm, cols_hbm, sp_hbm,
                    acc_sh, gbuf, idxg, idxs, zb, semA, semB):
    c = lax.axis_index("c")
    s = lax.axis_index("s")
    zeros16 = jnp.zeros((16,), jnp.float32)

    def zrow(i, carry):
        for q in range(D // 16):
            zb[i, pl.ds(q * 16, 16)] = zeros16
        return carry

    lax.fori_loop(0, 32, zrow, 0)
    for t in range(RPT // 32):
        pltpu.sync_copy(zb, acc_sh.at[pl.ds(s * RPT + t * 32, 32)])
    plsc.subcore_barrier()

    def run_block(st, n):
        pltpu.sync_copy(rowg_hbm.at[s, pl.ds(st, n)], idxg.at[pl.ds(0, n)])
        pltpu.sync_copy(cols_hbm.at[s, pl.ds(st, n)], idxs.at[pl.ds(0, n)])
        pltpu.async_copy(h_hbm.at[idxg.at[0]], gbuf.at[0], semA)
        pltpu.async_copy(h_hbm.at[idxg.at[1]], gbuf.at[1], semB)
        ng = n // 2

        def body(g, carry):
            j0 = 2 * g
            pltpu.make_async_copy(
                h_hbm.at[idxg.at[0]], gbuf.at[0], semA).wait()
            pltpu.sync_copy(gbuf.at[0], acc_sh.at[idxs.at[j0]], add=True)

            @pl.when(g < ng - 1)
            def _():
                pltpu.async_copy(h_hbm.at[idxg.at[j0 + 2]], gbuf.at[0], semA)

            pltpu.make_async_copy(
                h_hbm.at[idxg.at[1]], gbuf.at[1], semB).wait()
            pltpu.sync_copy(gbuf.at[1], acc_sh.at[idxs.at[j0 + 1]], add=True)

            @pl.when(g < ng - 1)
            def _():
                pltpu.async_copy(h_hbm.at[idxg.at[j0 + 3]], gbuf.at[1], semB)

            return carry

        lax.fori_loop(0, ng, body, 0)

    @pl.when(c == 0)
    def _():
        for st, n in _blocks(0, NCHUNK0):
            run_block(st, n)

    @pl.when(c == 1)
    def _():
        for st, n in _blocks(NCHUNK0, CPS - NCHUNK0):
            run_block(st, n)

    plsc.subcore_barrier()
    pltpu.sync_copy(acc_sh.at[pl.ds(s * RPT, RPT)],
                    sp_hbm.at[c, pl.ds(s * RPT, RPT)])


def _tc1_body(x_ref, w1_ref, degt_ref, h1p_ref, dinv_ref):
    deg = degt_ref[:, 0:1] + degt_ref[:, 1:2] + 0.5
    dinv = lax.rsqrt(deg)
    h = jnp.dot(x_ref[...], w1_ref[...], preferred_element_type=jnp.float32)
    h1p_ref[...] = h * dinv
    dinv_ref[...] = dinv


def _tc1(x, W1, degt):
    return pl.pallas_call(
        _tc1_body,
        grid=(GRID,),
        in_specs=[
            pl.BlockSpec((RB, D), lambda i: (i, 0)),
            pl.BlockSpec((D, D), lambda i: (0, 0)),
            pl.BlockSpec((RB, NC), lambda i: (i, 0)),
        ],
        out_specs=[
            pl.BlockSpec((RB, D), lambda i: (i, 0)),
            pl.BlockSpec((RB, 1), lambda i: (i, 0)),
        ],
        out_shape=[
            jax.ShapeDtypeStruct((N, D), jnp.float32),
            jax.ShapeDtypeStruct((N, 1), jnp.float32),
        ],
    )(x, W1, degt)


def _tc2_body(sp_ref, h1p_ref, dinv_ref, b1_ref, w2_ref, h2p_ref):
    ssum = sp_ref[0] + sp_ref[1]
    dinv = dinv_ref[...]
    z = jnp.maximum(dinv * (ssum + 0.5 * h1p_ref[...]) + b1_ref[...], 0.0)
    h2 = jnp.dot(z, w2_ref[...], preferred_element_type=jnp.float32)
    h2p_ref[...] = h2 * dinv


def _tc2(sp1, h1p, dinv, b1, W2):
    return pl.pallas_call(
        _tc2_body,
        grid=(GRID,),
        in_specs=[
            pl.BlockSpec((NC, RB, D), lambda i: (0, i, 0)),
            pl.BlockSpec((RB, D), lambda i: (i, 0)),
            pl.BlockSpec((RB, 1), lambda i: (i, 0)),
            pl.BlockSpec((1, D), lambda i: (0, 0)),
            pl.BlockSpec((D, D), lambda i: (0, 0)),
        ],
        out_specs=pl.BlockSpec((RB, D), lambda i: (i, 0)),
        out_shape=jax.ShapeDtypeStruct((N, D), jnp.float32),
    )(sp1, h1p, dinv, b1, W2)


def _tc3_body(sp_ref, h2p_ref, dinv_ref, b2_ref, out_ref):
    ssum = sp_ref[0] + sp_ref[1]
    out_ref[...] = dinv_ref[...] * (ssum + 0.5 * h2p_ref[...]) + b2_ref[...]


def _tc3(sp2, h2p, dinv, b2):
    return pl.pallas_call(
        _tc3_body,
        grid=(GRID,),
        in_specs=[
            pl.BlockSpec((NC, RB, D), lambda i: (0, i, 0)),
            pl.BlockSpec((RB, D), lambda i: (i, 0)),
            pl.BlockSpec((RB, 1), lambda i: (i, 0)),
            pl.BlockSpec((1, D), lambda i: (0, 0)),
        ],
        out_specs=pl.BlockSpec((RB, D), lambda i: (i, 0)),
        out_shape=jax.ShapeDtypeStruct((N, D), jnp.float32),
    )(sp2, h2p, dinv, b2)


@jax.jit
def kernel(x, edge_index, W1, b1, W2, b2):
    row = edge_index[0]
    col = edge_index[1]
    pad = EPAD - E
    dummy = DUMMY + (jnp.arange(pad, dtype=jnp.int32) % (ROWS - N))
    rowg = jnp.concatenate(
        [row, jnp.zeros((pad,), jnp.int32)]).reshape(NS, CPS, K)
    cols = jnp.concatenate([col, dummy]).reshape(NS, CPS, K)
    rowd = jnp.concatenate([row, dummy]).reshape(NS, CPS, K)

    degp = _deg_kernel(rowd)
    degt = degp.T
    h1p, dinv = _tc1(x, W1, degt)
    sp1 = _scatter_kernel(h1p, rowg, cols)
    h2p = _tc2(sp1, h1p, dinv, b1.reshape(1, D), W2)
    sp2 = _scatter_kernel(h2p, rowg, cols)
    return _tc3(sp2, h2p, dinv, b2.reshape(1, D))


# speedup vs baseline: 1.0779x; 1.0041x over previous
import functools

import jax
import jax.numpy as jnp
from jax import lax
from jax.experimental import pallas as pl
from jax.experimental.pallas import tpu as pltpu
from jax.experimental.pallas import tpu_sc as plsc

N = 10000
D = 128
E = 320000
NC = 2
NS = 16
K = 128
CPS = 160
EPAD = NS * CPS * K
NCHUNK0 = 64
IDXB = 32
ROWS = 10240
RPT = ROWS // NS
DUMMY = N
RB = 2000
GRID = N // RB

_mesh = plsc.VectorSubcoreMesh(core_axis_name="c", subcore_axis_name="s")


@functools.partial(
    pl.kernel,
    out_type=jax.ShapeDtypeStruct((NC, ROWS), jnp.float32),
    mesh=_mesh,
    scratch_types=[
        pltpu.VMEM_SHARED((ROWS,), jnp.float32),
        pltpu.VMEM((CPS // 2, K), jnp.int32),
        pltpu.VMEM((K,), jnp.float32),
        pltpu.VMEM((RPT,), jnp.float32),
    ],
)
def _deg_kernel(rowd_hbm, degp_hbm, deg_sh, idx_v, ones_v, zb_v):
    c = lax.axis_index("c")
    s = lax.axis_index("s")
    ones16 = jnp.ones((16,), jnp.float32)
    zeros16 = jnp.zeros((16,), jnp.float32)
    for q in range(K // 16):
        ones_v[pl.ds(q * 16, 16)] = ones16
    for q in range(RPT // 16):
        zb_v[pl.ds(q * 16, 16)] = zeros16
    pltpu.sync_copy(zb_v, deg_sh.at[pl.ds(s * RPT, RPT)])
    pltpu.sync_copy(rowd_hbm.at[s, pl.ds(c * (CPS // 2), CPS // 2)], idx_v)
    plsc.subcore_barrier()

    def body(j, carry):
        pltpu.sync_copy(ones_v, deg_sh.at[idx_v.at[j]], add=True)
        return carry

    lax.fori_loop(0, CPS // 2, body, 0)
    plsc.subcore_barrier()
    pltpu.sync_copy(deg_sh.at[pl.ds(s * RPT, RPT)],
                    degp_hbm.at[c, pl.ds(s * RPT, RPT)])


def _blocks(start, n):
    out = []
    while n > 0:
        b = min(n, IDXB)
        out.append((start, b))
        start += b
        n -= b
    return out


@functools.partial(
    pl.kernel,
    out_type=jax.ShapeDtypeStruct((NC, ROWS, D), jnp.float32),
    mesh=_mesh,
    scratch_types=[
        pltpu.VMEM_SHARED((ROWS, D), jnp.float32),
        pltpu.VMEM((2, K, D), jnp.float32),
        pltpu.VMEM((IDXB, K), jnp.int32),
        pltpu.VMEM((IDXB, K), jnp.int32),
        pltpu.VMEM((32, D), jnp.float32),
        pltpu.SemaphoreType.DMA,
        pltpu.SemaphoreType.DMA,
    ],
)
def _scatter_kernel(h_hbm, rowg_hbm, cols_hbm, sp_hbm,
                    acc_sh, gbuf, idxg, idxs, zb, semA, semB):
    c = lax.axis_index("c")
    s = lax.axis_index("s")
    zeros16 = jnp.zeros((16,), jnp.float32)

    def zrow(i, carry):
        for q in range(D // 16):
            zb[i, pl.ds(q * 16, 16)] = zeros16
        return carry

    lax.fori_loop(0, 32, zrow, 0)
    for t in range(RPT // 32):
        pltpu.sync_copy(zb, acc_sh.at[pl.ds(s * RPT + t * 32, 32)])
    plsc.subcore_barrier()

    def run_block(st, n):
        pltpu.sync_copy(rowg_hbm.at[s, pl.ds(st, n)], idxg.at[pl.ds(0, n)])
        pltpu.sync_copy(cols_hbm.at[s, pl.ds(st, n)], idxs.at[pl.ds(0, n)])
        pltpu.async_copy(h_hbm.at[idxg.at[0]], gbuf.at[0], semA)
        pltpu.async_copy(h_hbm.at[idxg.at[1]], gbuf.at[1], semB)
        ng = n // 2

        def body(g, carry):
            j0 = 2 * g
            pltpu.make_async_copy(
                h_hbm.at[idxg.at[0]], gbuf.at[0], semA).wait()
            pltpu.sync_copy(gbuf.at[0], acc_sh.at[idxs.at[j0]], add=True)

            @pl.when(g < ng - 1)
            def _():
                pltpu.async_copy(h_hbm.at[idxg.at[j0 + 2]], gbuf.at[0], semA)

            pltpu.make_async_copy(
                h_hbm.at[idxg.at[1]], gbuf.at[1], semB).wait()
            pltpu.sync_copy(gbuf.at[1], acc_sh.at[idxs.at[j0 + 1]], add=True)

            @pl.when(g < ng - 1)
            def _():
                pltpu.async_copy(h_hbm.at[idxg.at[j0 + 3]], gbuf.at[1], semB)

            return carry

        lax.fori_loop(0, ng, body, 0)

    @pl.when(c == 0)
    def _():
        for st, n in _blocks(0, NCHUNK0):
            run_block(st, n)

    @pl.when(c == 1)
    def _():
        for st, n in _blocks(NCHUNK0, CPS - NCHUNK0):
            run_block(st, n)

    plsc.subcore_barrier()
    pltpu.sync_copy(acc_sh.at[pl.ds(s * RPT, RPT)],
                    sp_hbm.at[c, pl.ds(s * RPT, RPT)])


def _tc1_body(x_ref, w1_ref, degt_ref, h1p_ref, dinv_ref):
    deg = degt_ref[:, 0:1] + degt_ref[:, 1:2] + 0.5
    dinv = lax.rsqrt(deg)
    h = jnp.dot(x_ref[...], w1_ref[...], preferred_element_type=jnp.float32)
    h1p_ref[...] = h * dinv
    dinv_ref[...] = dinv


def _tc1(x, W1, degt):
    return pl.pallas_call(
        _tc1_body,
        grid=(GRID,),
        in_specs=[
            pl.BlockSpec((RB, D), lambda i: (i, 0)),
            pl.BlockSpec((D, D), lambda i: (0, 0)),
            pl.BlockSpec((RB, NC), lambda i: (i, 0)),
        ],
        out_specs=[
            pl.BlockSpec((RB, D), lambda i: (i, 0)),
            pl.BlockSpec((RB, 1), lambda i: (i, 0)),
        ],
        out_shape=[
            jax.ShapeDtypeStruct((N, D), jnp.float32),
            jax.ShapeDtypeStruct((N, 1), jnp.float32),
        ],
    )(x, W1, degt)


def _tc2_body(sp_ref, h1p_ref, dinv_ref, b1_ref, w2_ref, h2p_ref):
    ssum = sp_ref[0] + sp_ref[1]
    dinv = dinv_ref[...]
    z = jnp.maximum(dinv * (ssum + 0.5 * h1p_ref[...]) + b1_ref[...], 0.0)
    h2 = jnp.dot(z, w2_ref[...], preferred_element_type=jnp.float32)
    h2p_ref[...] = h2 * dinv


def _tc2(sp1, h1p, dinv, b1, W2):
    return pl.pallas_call(
        _tc2_body,
        grid=(GRID,),
        in_specs=[
            pl.BlockSpec((NC, RB, D), lambda i: (0, i, 0)),
            pl.BlockSpec((RB, D), lambda i: (i, 0)),
            pl.BlockSpec((RB, 1), lambda i: (i, 0)),
            pl.BlockSpec((1, D), lambda i: (0, 0)),
            pl.BlockSpec((D, D), lambda i: (0, 0)),
        ],
        out_specs=pl.BlockSpec((RB, D), lambda i: (i, 0)),
        out_shape=jax.ShapeDtypeStruct((N, D), jnp.float32),
    )(sp1, h1p, dinv, b1, W2)


def _tc3_body(sp_ref, h2p_ref, dinv_ref, b2_ref, out_ref):
    ssum = sp_ref[0] + sp_ref[1]
    out_ref[...] = dinv_ref[...] * (ssum + 0.5 * h2p_ref[...]) + b2_ref[...]


def _tc3(sp2, h2p, dinv, b2):
    return pl.pallas_call(
        _tc3_body,
        grid=(GRID,),
        in_specs=[
            pl.BlockSpec((NC, RB, D), lambda i: (0, i, 0)),
            pl.BlockSpec((RB, D), lambda i: (i, 0)),
            pl.BlockSpec((RB, 1), lambda i: (i, 0)),
            pl.BlockSpec((1, D), lambda i: (0, 0)),
        ],
        out_specs=pl.BlockSpec((RB, D), lambda i: (i, 0)),
        out_shape=jax.ShapeDtypeStruct((N, D), jnp.float32),
    )(sp2, h2p, dinv, b2)


@jax.jit
def kernel(x, edge_index, W1, b1, W2, b2):
    row = edge_index[0]
    col = edge_index[1]
    pad = EPAD - E
    dummy = DUMMY + (jnp.arange(pad, dtype=jnp.int32) % (ROWS - N))
    rowg = jnp.concatenate(
        [row, jnp.zeros((pad,), jnp.int32)]).reshape(NS, CPS, K)
    cols = jnp.concatenate([col, dummy]).reshape(NS, CPS, K)
    rowd = jnp.concatenate([row, dummy]).reshape(NS, CPS, K)

    degp = _deg_kernel(rowd)
    degt = degp.T
    h1p, dinv = _tc1(x, W1, degt)
    sp1 = _scatter_kernel(h1p, rowg, cols)
    h2p = _tc2(sp1, h1p, dinv, b1.reshape(1, D), W2)
    sp2 = _scatter_kernel(h2p, rowg, cols)
    return _tc3(sp2, h2p, dinv, b2.reshape(1, D))
